# baseline (speedup 1.0000x reference)
import functools

import jax
import jax.numpy as jnp
from jax import lax
from jax.experimental import pallas as pl
from jax.experimental.pallas import tpu as pltpu
from jax.experimental.pallas import tpu_sc as plsc

_NC = 2
_NS = 16
_NW = _NC * _NS

_CHUNK = 512

_ROWS_W = 31248
_PCHUNK = 496
_NPCH = _ROWS_W // _PCHUNK
_LANES = 16


def _sc_pack_pairs(table):
    vocab, embed = table.shape
    mesh = plsc.VectorSubcoreMesh(core_axis_name="c", subcore_axis_name="s")
    tail = vocab - _NW * _ROWS_W

    @functools.partial(
        pl.kernel,
        mesh=mesh,
        out_type=jax.ShapeDtypeStruct((vocab // 2, 2 * embed), jnp.float32),
        scratch_types=[
            pltpu.VMEM((_PCHUNK, embed), jnp.float32),
            pltpu.VMEM((_PCHUNK // 2, 2 * embed), jnp.float32),
        ],
    )
    def pack_kernel(table_hbm, out_hbm, buf_a, buf_b):
        wid = lax.axis_index("s") * _NC + lax.axis_index("c")
        base = wid * _ROWS_W

        def pack_chunk(a, nrows):
            a = pl.multiple_of(a, 8)
            pltpu.sync_copy(table_hbm.at[pl.ds(a, nrows)], buf_a.at[pl.ds(0, nrows)])

            @pl.loop(0, nrows // 2)
            def _(m):
                for half in range(2):
                    for l in range(embed // _LANES):
                        src = (pl.ds(2 * m + half, 1), pl.ds(_LANES * l, _LANES))
                        dst = (pl.ds(m, 1),
                               pl.ds(embed * half + _LANES * l, _LANES))
                        buf_b.at[dst][...] = buf_a.at[src][...]

            pltpu.sync_copy(
                buf_b.at[pl.ds(0, nrows // 2)],
                out_hbm.at[pl.ds(pl.multiple_of(a // 2, 8), nrows // 2)],
            )

        @pl.loop(0, _NPCH)
        def _(c):
            pack_chunk(base + c * _PCHUNK, _PCHUNK)

        @pl.when(wid == _NW - 1)
        def _():
            pack_chunk(_NW * _ROWS_W, tail)

    return pack_kernel(table)


def _sc_gather(table128, idx):
    n, = idx.shape
    d = table128.shape[1]
    b_per_w = n // _NW
    n_chunks = b_per_w // _CHUNK
    mesh = plsc.VectorSubcoreMesh(core_axis_name="c", subcore_axis_name="s")

    @functools.partial(
        pl.kernel,
        mesh=mesh,
        out_type=jax.ShapeDtypeStruct((n, d), jnp.float32),
        scratch_types=[
            pltpu.VMEM((b_per_w,), jnp.int32),
            pltpu.VMEM((_CHUNK, d), jnp.float32),
            pltpu.SemaphoreType.DMA,
        ],
    )
    def gather_kernel(table_hbm, idx_hbm, out_hbm, idx_v, rows_v, sem):
        wid = lax.axis_index("s") * _NC + lax.axis_index("c")
        base = wid * b_per_w
        pltpu.sync_copy(idx_hbm.at[pl.ds(base, b_per_w)], idx_v)

        @pl.loop(0, n_chunks)
        def _(c):
            off = c * _CHUNK
            pltpu.async_copy(
                table_hbm.at[idx_v.at[pl.ds(off, _CHUNK)]], rows_v, sem
            ).wait()
            pltpu.sync_copy(rows_v, out_hbm.at[pl.ds(base + off, _CHUNK)])

    return gather_kernel(table128, idx)


def _mlp_block(rows_ref, par_ref, w1_ref, b1_ref, w2_ref, b2_ref, o_ref):
    window = rows_ref.shape[0]
    embed = rows_ref.shape[2] // 2
    h = b1_ref[...]
    for w in range(window):
        rw = rows_ref[w]
        p = par_ref[w][:, None]
        sel = jnp.where(p == 1, rw[:, embed:], rw[:, :embed])
        h = h + jnp.dot(sel, w1_ref[w], preferred_element_type=jnp.float32)
    h = jnp.maximum(h, 0.0)
    o_ref[...] = (
        jnp.dot(h, w2_ref[...], preferred_element_type=jnp.float32) + b2_ref[...]
    )


def _tc_mlp(rows, parity, w1s, b1, w2, b2, block_b=1024):
    window, batch, d2 = rows.shape
    embed = d2 // 2
    hidden = w2.shape[0]
    out_dim = w2.shape[1]
    return pl.pallas_call(
        _mlp_block,
        grid=(batch // block_b,),
        in_specs=[
            pl.BlockSpec((window, block_b, d2), lambda i: (0, i, 0)),
            pl.BlockSpec((window, block_b), lambda i: (0, i)),
            pl.BlockSpec((window, embed, hidden), lambda i: (0, 0, 0)),
            pl.BlockSpec((1, hidden), lambda i: (0, 0)),
            pl.BlockSpec((hidden, out_dim), lambda i: (0, 0)),
            pl.BlockSpec((1, out_dim), lambda i: (0, 0)),
        ],
        out_specs=pl.BlockSpec((block_b, out_dim), lambda i: (i, 0)),
        out_shape=jax.ShapeDtypeStruct((batch, out_dim), jnp.float32),
    )(rows, parity, w1s, b1, w2, b2)


def kernel(word_indices, table, W1, b1, W2, b2):
    batch, window = word_indices.shape
    vocab, embed = table.shape

    table128 = _sc_pack_pairs(table)

    idx_wmajor = word_indices.T.reshape(-1).astype(jnp.int32)
    idx_half = idx_wmajor >> 1
    parity = (idx_wmajor & 1).reshape(window, batch)

    rows = _sc_gather(table128, idx_half)
    rows = rows.reshape(window, batch, 2 * embed)

    w1s = W1.reshape(window, embed, -1)
    out = _tc_mlp(rows, parity, w1s, b1.reshape(1, -1), W2, b2.reshape(1, -1))
    return out


# speedup vs baseline: 3.3632x; 3.3632x over previous
import functools

import jax
import jax.numpy as jnp
from jax import lax
from jax.experimental import pallas as pl
from jax.experimental.pallas import tpu as pltpu
from jax.experimental.pallas import tpu_sc as plsc

_NC = 2
_NS = 16
_NW = _NC * _NS

_CHUNK = 512
_KV = 8192


def _tpack_block(t_ref, o_ref):
    y = t_ref[...].T
    o_ref[...] = jnp.concatenate([y, jnp.zeros_like(y)], axis=1)


def _tc_transpose_pack(table_t):
    embed, vocab = table_t.shape
    return pl.pallas_call(
        _tpack_block,
        grid=(pl.cdiv(vocab, _KV),),
        in_specs=[pl.BlockSpec((embed, _KV), lambda i: (0, i))],
        out_specs=pl.BlockSpec((_KV, 2 * embed), lambda i: (i, 0)),
        out_shape=jax.ShapeDtypeStruct((vocab, 2 * embed), jnp.float32),
    )(table_t)


def _sc_gather(table_wide, idx):
    n, = idx.shape
    d = table_wide.shape[1]
    b_per_w = n // _NW
    n_chunks = b_per_w // _CHUNK
    mesh = plsc.VectorSubcoreMesh(core_axis_name="c", subcore_axis_name="s")

    @functools.partial(
        pl.kernel,
        mesh=mesh,
        out_type=jax.ShapeDtypeStruct((n, d), jnp.float32),
        scratch_types=[
            pltpu.VMEM((b_per_w,), jnp.int32),
            pltpu.VMEM((_CHUNK, d), jnp.float32),
            pltpu.SemaphoreType.DMA,
        ],
    )
    def gather_kernel(table_hbm, idx_hbm, out_hbm, idx_v, rows_v, sem):
        wid = lax.axis_index("s") * _NC + lax.axis_index("c")
        base = wid * b_per_w
        pltpu.sync_copy(idx_hbm.at[pl.ds(base, b_per_w)], idx_v)

        @pl.loop(0, n_chunks)
        def _(c):
            off = c * _CHUNK
            pltpu.async_copy(
                table_hbm.at[idx_v.at[pl.ds(off, _CHUNK)]], rows_v, sem
            ).wait()
            pltpu.sync_copy(rows_v, out_hbm.at[pl.ds(base + off, _CHUNK)])

    return gather_kernel(table_wide, idx)


def _mlp_block(r0, r1, r2, r3, r4, w1_ref, b1_ref, w2_ref, b2_ref, o_ref):
    h = b1_ref[...]
    embed = w1_ref.shape[1]
    for w, rw in enumerate((r0, r1, r2, r3, r4)):
        h = h + jnp.dot(rw[:, :embed], w1_ref[w],
                        preferred_element_type=jnp.float32)
    h = jnp.maximum(h, 0.0)
    o_ref[...] = (
        jnp.dot(h, w2_ref[...], preferred_element_type=jnp.float32) + b2_ref[...]
    )


def _tc_mlp(rows, w1s, b1, w2, b2, batch, block_b=1024):
    window, embed, hidden = w1s.shape
    out_dim = w2.shape[1]
    nb = batch // block_b
    row_specs = [
        pl.BlockSpec((block_b, 2 * embed), functools.partial(
            lambda w, i: (w * nb + i, 0), w))
        for w in range(window)
    ]
    return pl.pallas_call(
        _mlp_block,
        grid=(nb,),
        in_specs=row_specs + [
            pl.BlockSpec((window, embed, hidden), lambda i: (0, 0, 0)),
            pl.BlockSpec((1, hidden), lambda i: (0, 0)),
            pl.BlockSpec((hidden, out_dim), lambda i: (0, 0)),
            pl.BlockSpec((1, out_dim), lambda i: (0, 0)),
        ],
        out_specs=pl.BlockSpec((block_b, out_dim), lambda i: (i, 0)),
        out_shape=jax.ShapeDtypeStruct((batch, out_dim), jnp.float32),
    )(*([rows] * window), w1s, b1, w2, b2)


def kernel(word_indices, table, W1, b1, W2, b2):
    batch, window = word_indices.shape
    vocab, embed = table.shape

    table_wide = _tc_transpose_pack(table.T)

    idx_wmajor = word_indices.T.reshape(-1).astype(jnp.int32)

    rows = _sc_gather(table_wide, idx_wmajor)

    w1s = W1.reshape(window, embed, -1)
    return _tc_mlp(rows, w1s, b1.reshape(1, -1), W2, b2.reshape(1, -1), batch)


# speedup vs baseline: 3.5518x; 1.0561x over previous
import functools

import jax
import jax.numpy as jnp
from jax import lax
from jax.experimental import pallas as pl
from jax.experimental.pallas import tpu as pltpu
from jax.experimental.pallas import tpu_sc as plsc

_NC = 2
_NS = 16
_NW = _NC * _NS

_CHUNK = 512
_KV = 16384


def _tpack_block(t_ref, o_ref):
    y = t_ref[...].T
    o_ref[...] = jnp.concatenate([y, jnp.zeros_like(y)], axis=1)


def _tc_transpose_pack(table_t):
    embed, vocab = table_t.shape
    return pl.pallas_call(
        _tpack_block,
        grid=(pl.cdiv(vocab, _KV),),
        in_specs=[pl.BlockSpec((embed, _KV), lambda i: (0, i))],
        out_specs=pl.BlockSpec((_KV, 2 * embed), lambda i: (i, 0)),
        out_shape=jax.ShapeDtypeStruct((vocab, 2 * embed), jnp.float32),
    )(table_t)


def _sc_gather(table_wide, idx):
    n, = idx.shape
    d = table_wide.shape[1]
    b_per_w = n // _NW
    n_chunks = b_per_w // _CHUNK
    mesh = plsc.VectorSubcoreMesh(core_axis_name="c", subcore_axis_name="s")

    @functools.partial(
        pl.kernel,
        mesh=mesh,
        out_type=jax.ShapeDtypeStruct((n, d), jnp.float32),
        scratch_types=[
            pltpu.VMEM((b_per_w,), jnp.int32),
            pltpu.VMEM((_CHUNK, d), jnp.float32),
            pltpu.SemaphoreType.DMA,
        ],
    )
    def gather_kernel(table_hbm, idx_hbm, out_hbm, idx_v, rows_v, sem):
        wid = lax.axis_index("s") * _NC + lax.axis_index("c")
        base = wid * b_per_w
        pltpu.sync_copy(idx_hbm.at[pl.ds(base, b_per_w)], idx_v)

        @pl.loop(0, n_chunks)
        def _(c):
            off = c * _CHUNK
            pltpu.async_copy(
                table_hbm.at[idx_v.at[pl.ds(off, _CHUNK)]], rows_v, sem
            ).wait()
            pltpu.sync_copy(rows_v, out_hbm.at[pl.ds(base + off, _CHUNK)])

    return gather_kernel(table_wide, idx)


def _mlp_block(r0, r1, r2, r3, r4, w1_ref, b1_ref, w2_ref, b2_ref, o_ref):
    h = b1_ref[...]
    embed = w1_ref.shape[1]
    for w, rw in enumerate((r0, r1, r2, r3, r4)):
        h = h + jnp.dot(rw[:, :embed], w1_ref[w],
                        preferred_element_type=jnp.float32)
    h = jnp.maximum(h, 0.0)
    o_ref[...] = (
        jnp.dot(h, w2_ref[...], preferred_element_type=jnp.float32) + b2_ref[...]
    )


def _tc_mlp(rows, w1s, b1, w2, b2, batch, block_b=1024):
    window, embed, hidden = w1s.shape
    out_dim = w2.shape[1]
    nb = batch // block_b
    row_specs = [
        pl.BlockSpec((block_b, 2 * embed), functools.partial(
            lambda w, i: (w * nb + i, 0), w))
        for w in range(window)
    ]
    return pl.pallas_call(
        _mlp_block,
        grid=(nb,),
        in_specs=row_specs + [
            pl.BlockSpec((window, embed, hidden), lambda i: (0, 0, 0)),
            pl.BlockSpec((1, hidden), lambda i: (0, 0)),
            pl.BlockSpec((hidden, out_dim), lambda i: (0, 0)),
            pl.BlockSpec((1, out_dim), lambda i: (0, 0)),
        ],
        out_specs=pl.BlockSpec((block_b, out_dim), lambda i: (i, 0)),
        out_shape=jax.ShapeDtypeStruct((batch, out_dim), jnp.float32),
    )(*([rows] * window), w1s, b1, w2, b2)


def kernel(word_indices, table, W1, b1, W2, b2):
    batch, window = word_indices.shape
    vocab, embed = table.shape

    table_wide = _tc_transpose_pack(table.T)

    idx_wmajor = word_indices.T.reshape(-1).astype(jnp.int32)

    rows = _sc_gather(table_wide, idx_wmajor)

    w1s = W1.reshape(window, embed, -1)
    return _tc_mlp(rows, w1s, b1.reshape(1, -1), W2, b2.reshape(1, -1), batch)


# speedup vs baseline: 3.6346x; 1.0233x over previous
import functools

import jax
import jax.numpy as jnp
from jax import lax
from jax.experimental import pallas as pl
from jax.experimental.pallas import tpu as pltpu
from jax.experimental.pallas import tpu_sc as plsc

_NC = 2
_NS = 16
_NW = _NC * _NS

_CHUNK = 512
_KV = 32768


def _tpack_block(t_ref, o_ref):
    y = t_ref[...].T
    o_ref[...] = jnp.concatenate([y, jnp.zeros_like(y)], axis=1)


def _tc_transpose_pack(table_t):
    embed, vocab = table_t.shape
    return pl.pallas_call(
        _tpack_block,
        grid=(pl.cdiv(vocab, _KV),),
        in_specs=[pl.BlockSpec((embed, _KV), lambda i: (0, i))],
        out_specs=pl.BlockSpec((_KV, 2 * embed), lambda i: (i, 0)),
        out_shape=jax.ShapeDtypeStruct((vocab, 2 * embed), jnp.float32),
    )(table_t)


def _sc_gather(table_wide, idx):
    n, = idx.shape
    d = table_wide.shape[1]
    b_per_w = n // _NW
    n_chunks = b_per_w // _CHUNK
    mesh = plsc.VectorSubcoreMesh(core_axis_name="c", subcore_axis_name="s")

    @functools.partial(
        pl.kernel,
        mesh=mesh,
        out_type=jax.ShapeDtypeStruct((n, d), jnp.float32),
        scratch_types=[
            pltpu.VMEM((b_per_w,), jnp.int32),
            pltpu.VMEM((_CHUNK, d), jnp.float32),
            pltpu.SemaphoreType.DMA,
        ],
    )
    def gather_kernel(table_hbm, idx_hbm, out_hbm, idx_v, rows_v, sem):
        wid = lax.axis_index("s") * _NC + lax.axis_index("c")
        base = wid * b_per_w
        pltpu.sync_copy(idx_hbm.at[pl.ds(base, b_per_w)], idx_v)

        @pl.loop(0, n_chunks)
        def _(c):
            off = c * _CHUNK
            pltpu.async_copy(
                table_hbm.at[idx_v.at[pl.ds(off, _CHUNK)]], rows_v, sem
            ).wait()
            pltpu.sync_copy(rows_v, out_hbm.at[pl.ds(base + off, _CHUNK)])

    return gather_kernel(table_wide, idx)


def _mlp_block(r0, r1, r2, r3, r4, w1_ref, b1_ref, w2_ref, b2_ref, o_ref):
    h = b1_ref[...]
    embed = w1_ref.shape[1]
    for w, rw in enumerate((r0, r1, r2, r3, r4)):
        h = h + jnp.dot(rw[:, :embed], w1_ref[w],
                        preferred_element_type=jnp.float32)
    h = jnp.maximum(h, 0.0)
    o_ref[...] = (
        jnp.dot(h, w2_ref[...], preferred_element_type=jnp.float32) + b2_ref[...]
    )


def _tc_mlp(rows, w1s, b1, w2, b2, batch, block_b=1024):
    window, embed, hidden = w1s.shape
    out_dim = w2.shape[1]
    nb = batch // block_b
    row_specs = [
        pl.BlockSpec((block_b, 2 * embed), functools.partial(
            lambda w, i: (w * nb + i, 0), w))
        for w in range(window)
    ]
    return pl.pallas_call(
        _mlp_block,
        grid=(nb,),
        in_specs=row_specs + [
            pl.BlockSpec((window, embed, hidden), lambda i: (0, 0, 0)),
            pl.BlockSpec((1, hidden), lambda i: (0, 0)),
            pl.BlockSpec((hidden, out_dim), lambda i: (0, 0)),
            pl.BlockSpec((1, out_dim), lambda i: (0, 0)),
        ],
        out_specs=pl.BlockSpec((block_b, out_dim), lambda i: (i, 0)),
        out_shape=jax.ShapeDtypeStruct((batch, out_dim), jnp.float32),
    )(*([rows] * window), w1s, b1, w2, b2)


def kernel(word_indices, table, W1, b1, W2, b2):
    batch, window = word_indices.shape
    vocab, embed = table.shape

    table_wide = _tc_transpose_pack(table.T)

    idx_wmajor = word_indices.T.reshape(-1).astype(jnp.int32)

    rows = _sc_gather(table_wide, idx_wmajor)

    w1s = W1.reshape(window, embed, -1)
    return _tc_mlp(rows, w1s, b1.reshape(1, -1), W2, b2.reshape(1, -1), batch)
